# baseline (speedup 1.0000x reference)
import functools

import jax
import jax.numpy as jnp
from jax import lax
from jax.experimental import pallas as pl
from jax.experimental.pallas import tpu as pltpu
from jax.experimental.pallas import tpu_sc as plsc

VOCAB = 1000
VPAD = 1024
NTILE = VPAD // 128
TAIL = VOCAB - 128 * (NTILE - 1)
NTOK = 1024 * 50
NH = 2
NTOKH = NTOK // NH
NC, NS, L = 2, 16, 16
NW = NC * NS
RPT = NTOKH // NW
RPAD = 1024
CH = 16
NCHUNK = RPT // CH
NP = NCHUNK // 2


def _lse_body(table_ref, out_ref):
    x = table_ref[...]
    m = jnp.max(x, axis=1, keepdims=True)
    s = jnp.sum(jnp.exp(x - m), axis=1, keepdims=True)
    lse = m + jnp.log(s)
    out_ref[...] = jnp.concatenate(
        [lse, jnp.zeros((VPAD - VOCAB, 1), jnp.float32)], axis=0)


def _loss_body(p1_ref, p2_ref, out_ref):
    out_ref[...] = ((jnp.sum(p1_ref[...]) + jnp.sum(p2_ref[...]))
                    / NTOK).reshape(1, 1)


TR = 512


def _tr_body(in_ref, tails_ref, out_ref):
    x = in_ref[...]
    t = tails_ref[...]
    full = jnp.concatenate([x, t[:, :TAIL]], axis=1)
    out_ref[...] = full.T


def _tr2_body(in_ref, tails_ref, prev_ref, out_ref):
    del prev_ref
    _tr_body(in_ref, tails_ref, out_ref)


def _sc_body(table_hbm, idx_hbm, tgt_hbm, lse_hbm, out_hbm, tails_hbm,
             part_hbm, idx_v, tgt_v, lse_v, buf0, buf1, acc_v,
             gsem0, gsem1, ssem0, ssem1):
    wid = lax.axis_index("s") * NC + lax.axis_index("c")
    rbase = wid * RPT
    pltpu.sync_copy(idx_hbm.at[pl.ds(wid * RPAD, RPAD)], idx_v)
    pltpu.sync_copy(tgt_hbm.at[pl.ds(wid * RPAD, RPAD)], tgt_v)
    pltpu.sync_copy(lse_hbm, lse_v)

    lane = lax.iota(jnp.int32, L)

    def gather(c, buf, sem):
        return pltpu.make_async_copy(
            table_hbm.at[idx_v.at[pl.ds(c * CH, CH)]], buf, sem)

    def scatters(c, buf, sem):
        r0 = rbase + c * CH
        cps = []
        for t in range(NTILE - 1):
            cps.append(pltpu.make_async_copy(
                buf.at[pl.ds(0, CH), pl.ds(t * 128, 128)],
                out_hbm.at[pl.ds(r0, CH), pl.ds(t * 128, 128)], sem))
        cps.append(pltpu.make_async_copy(
            buf.at[pl.ds(0, CH), pl.ds((NTILE - 1) * 128, 128)],
            tails_hbm.at[pl.ds(r0, CH)], sem))
        return cps

    def start_scatters(c, buf, sem):
        for cp in scatters(c, buf, sem):
            cp.start()

    def wait_scatters(c, buf, sem):
        for cp in scatters(c, buf, sem):
            cp.wait()

    def loss(c, buf, acc):
        for j in range(CH // L):
            lo = c * CH + j * L
            v = idx_v[pl.ds(lo, L)]
            tg = tgt_v[pl.ds(lo, L)]
            trow = lane + j * L
            pick = plsc.load_gather(buf, [trow, tg])
            lseg = plsc.load_gather(lse_v, [v])
            acc = acc + (lseg - pick)
        return acc

    gather(0, buf0, gsem0).start()

    def step(k, acc):
        a = 2 * k
        b = a + 1
        gather(a, buf0, gsem0).wait()

        @pl.when(k > 0)
        def _():
            wait_scatters(b - 2, buf1, ssem1)

        gather(b, buf1, gsem1).start()
        acc = loss(a, buf0, acc)
        start_scatters(a, buf0, ssem0)
        gather(b, buf1, gsem1).wait()

        @pl.when(k < NP - 1)
        def _():
            wait_scatters(a, buf0, ssem0)
            gather(a + 2, buf0, gsem0).start()

        acc = loss(b, buf1, acc)
        start_scatters(b, buf1, ssem1)
        return acc

    acc = lax.fori_loop(0, NP, step, jnp.zeros((L,), jnp.float32))
    wait_scatters(2 * NP - 2, buf0, ssem0)
    wait_scatters(2 * NP - 1, buf1, ssem1)
    zero = jnp.zeros((L,), jnp.float32)
    for j in range(8):
        acc_v[pl.ds(j * L, L)] = acc if j == 0 else zero
    pltpu.sync_copy(acc_v, part_hbm.at[pl.ds(wid * 128, 128)])


_sc_gather = functools.partial(
    pl.kernel,
    out_type=[jax.ShapeDtypeStruct((NTOKH, VOCAB), jnp.float32),
              jax.ShapeDtypeStruct((NTOKH, 128), jnp.float32),
              jax.ShapeDtypeStruct((NW * 128,), jnp.float32)],
    mesh=plsc.VectorSubcoreMesh(core_axis_name="c", subcore_axis_name="s"),
    compiler_params=pltpu.CompilerParams(needs_layout_passes=False,
                                         use_tc_tiling_on_sc=True),
    scratch_types=[
        pltpu.VMEM((RPAD,), jnp.int32),
        pltpu.VMEM((RPAD,), jnp.int32),
        pltpu.VMEM((VPAD,), jnp.float32),
        pltpu.VMEM((CH, VPAD), jnp.float32),
        pltpu.VMEM((CH, VPAD), jnp.float32),
        pltpu.VMEM((128,), jnp.float32),
        pltpu.SemaphoreType.DMA,
        pltpu.SemaphoreType.DMA,
        pltpu.SemaphoreType.DMA,
        pltpu.SemaphoreType.DMA,
    ],
)(_sc_body)


def kernel(table, idx, targets):
    table_p = jnp.pad(table, ((0, 0), (0, VPAD - VOCAB)))
    idx_flat = idx.reshape(-1)
    tgt_flat = targets.reshape(-1)
    lse = pl.pallas_call(
        _lse_body,
        out_shape=jax.ShapeDtypeStruct((VPAD, 1), jnp.float32),
    )(table).reshape(-1)

    halves = []
    for h in range(NH):
        sl = slice(h * NTOKH, (h + 1) * NTOKH)
        idx_pad = jnp.pad(idx_flat[sl].reshape(NW, RPT),
                          ((0, 0), (0, RPAD - RPT)))
        tgt_pad = jnp.pad(tgt_flat[sl].reshape(NW, RPT),
                          ((0, 0), (0, RPAD - RPT)))
        halves.append(_sc_gather(table_p, idx_pad.reshape(-1),
                                 tgt_pad.reshape(-1), lse))

    nblk = NTOKH // TR
    logits_t = pl.pallas_call(
        _tr_body,
        grid=(nblk,),
        in_specs=[pl.BlockSpec((TR, 128 * (NTILE - 1)), lambda i: (i, 0)),
                  pl.BlockSpec((TR, 128), lambda i: (i, 0))],
        out_specs=pl.BlockSpec((VOCAB, TR), lambda i: (0, i)),
        out_shape=jax.ShapeDtypeStruct((VOCAB, NTOK), jnp.float32),
    )(halves[0][0], halves[0][1])
    logits_t = pl.pallas_call(
        _tr2_body,
        grid=(nblk,),
        in_specs=[pl.BlockSpec((TR, 128 * (NTILE - 1)), lambda i: (i, 0)),
                  pl.BlockSpec((TR, 128), lambda i: (i, 0)),
                  pl.BlockSpec(memory_space=pl.ANY)],
        out_specs=pl.BlockSpec((VOCAB, TR), lambda i: (0, i + nblk)),
        out_shape=jax.ShapeDtypeStruct((VOCAB, NTOK), jnp.float32),
        input_output_aliases={2: 0},
    )(halves[1][0], halves[1][1], logits_t)
    logits = logits_t.T
    loss = pl.pallas_call(
        _loss_body,
        out_shape=jax.ShapeDtypeStruct((1, 1), jnp.float32),
    )(halves[0][2].reshape(NW, 128), halves[1][2].reshape(NW, 128))[0, 0]
    return logits, loss


# speedup vs baseline: 1.0216x; 1.0216x over previous
import functools

import jax
import jax.numpy as jnp
from jax import lax
from jax.experimental import pallas as pl
from jax.experimental.pallas import tpu as pltpu
from jax.experimental.pallas import tpu_sc as plsc

VOCAB = 1000
VPAD = 1024
NTILE = VPAD // 128
TAIL = VOCAB - 128 * (NTILE - 1)
NTOK = 1024 * 50
NC, NS, L = 2, 16, 16
NW = NC * NS
RPT = NTOK // NW
RPAD = 2048
CH = 16
NCHUNK = RPT // CH
NQ = NCHUNK // 4


def _lse_body(table_ref, out_ref):
    x = table_ref[...]
    m = jnp.max(x, axis=1, keepdims=True)
    s = jnp.sum(jnp.exp(x - m), axis=1, keepdims=True)
    lse = m + jnp.log(s)
    out_ref[...] = jnp.concatenate(
        [lse, jnp.zeros((VPAD - VOCAB, 1), jnp.float32)], axis=0)


def _loss_body(part_ref, out_ref):
    out_ref[...] = (jnp.sum(part_ref[...]) / NTOK).reshape(1, 1)


def _sc_body(table_hbm, idx_hbm, tgt_hbm, lse_hbm, out_hbm, tails_hbm,
             part_hbm, idx_v, tgt_v, lse_v, buf0, buf1, buf2, buf3, acc_v,
             gsem0, gsem1, gsem2, gsem3, ssem0, ssem1, ssem2, ssem3):
    wid = lax.axis_index("s") * NC + lax.axis_index("c")
    rbase = wid * RPT
    pltpu.sync_copy(idx_hbm.at[pl.ds(wid * RPAD, RPAD)], idx_v)
    pltpu.sync_copy(tgt_hbm.at[pl.ds(wid * RPAD, RPAD)], tgt_v)
    pltpu.sync_copy(lse_hbm, lse_v)

    lane = lax.iota(jnp.int32, L)

    def gather(c, buf, sem):
        return pltpu.make_async_copy(
            table_hbm.at[idx_v.at[pl.ds(c * CH, CH)]], buf, sem)

    def scatters(c, buf, sem):
        r0 = rbase + c * CH
        cps = []
        for t in range(NTILE - 1):
            cps.append(pltpu.make_async_copy(
                buf.at[pl.ds(0, CH), pl.ds(t * 128, 128)],
                out_hbm.at[pl.ds(r0, CH), pl.ds(t * 128, 128)], sem))
        cps.append(pltpu.make_async_copy(
            buf.at[pl.ds(0, CH), pl.ds((NTILE - 1) * 128, 128)],
            tails_hbm.at[pl.ds(r0, CH)], sem))
        return cps

    def start_scatters(c, buf, sem):
        for cp in scatters(c, buf, sem):
            cp.start()

    def wait_scatters(c, buf, sem):
        for cp in scatters(c, buf, sem):
            cp.wait()

    def loss(c, buf, acc):
        for j in range(CH // L):
            lo = c * CH + j * L
            v = idx_v[pl.ds(lo, L)]
            tg = tgt_v[pl.ds(lo, L)]
            trow = lane + j * L
            pick = plsc.load_gather(buf, [trow, tg])
            lseg = plsc.load_gather(lse_v, [v])
            acc = acc + (lseg - pick)
        return acc

    bufs = (buf0, buf1, buf2, buf3)
    gsems = (gsem0, gsem1, gsem2, gsem3)
    ssems = (ssem0, ssem1, ssem2, ssem3)

    gather(0, buf0, gsem0).start()
    gather(1, buf1, gsem1).start()

    def step(k, acc):
        for i in range(4):
            c = 4 * k + i
            n = (i + 2) % 4
            gather(c, bufs[i], gsems[i]).wait()
            if i < 2:
                @pl.when(k > 0)
                def _(c=c, n=n):
                    wait_scatters(c - 2, bufs[n], ssems[n])
                gather(c + 2, bufs[n], gsems[n]).start()
            else:
                wait_scatters(c - 2, bufs[n], ssems[n])

                @pl.when(k < NQ - 1)
                def _(c=c, n=n):
                    gather(c + 2, bufs[n], gsems[n]).start()
            acc = loss(c, bufs[i], acc)
            start_scatters(c, bufs[i], ssems[i])
        return acc

    acc = lax.fori_loop(0, NQ, step, jnp.zeros((L,), jnp.float32))
    wait_scatters(NCHUNK - 2, buf2, ssem2)
    wait_scatters(NCHUNK - 1, buf3, ssem3)
    zero = jnp.zeros((L,), jnp.float32)
    for j in range(8):
        acc_v[pl.ds(j * L, L)] = acc if j == 0 else zero
    pltpu.sync_copy(acc_v, part_hbm.at[pl.ds(wid * 128, 128)])


_sc_gather = functools.partial(
    pl.kernel,
    out_type=[jax.ShapeDtypeStruct((NTOK, VOCAB), jnp.float32),
              jax.ShapeDtypeStruct((NTOK, 128), jnp.float32),
              jax.ShapeDtypeStruct((NW * 128,), jnp.float32)],
    mesh=plsc.VectorSubcoreMesh(core_axis_name="c", subcore_axis_name="s"),
    compiler_params=pltpu.CompilerParams(needs_layout_passes=False,
                                         use_tc_tiling_on_sc=True),
    scratch_types=[
        pltpu.VMEM((RPAD,), jnp.int32),
        pltpu.VMEM((RPAD,), jnp.int32),
        pltpu.VMEM((VPAD,), jnp.float32),
        pltpu.VMEM((CH, VPAD), jnp.float32),
        pltpu.VMEM((CH, VPAD), jnp.float32),
        pltpu.VMEM((CH, VPAD), jnp.float32),
        pltpu.VMEM((CH, VPAD), jnp.float32),
        pltpu.VMEM((128,), jnp.float32),
        pltpu.SemaphoreType.DMA,
        pltpu.SemaphoreType.DMA,
        pltpu.SemaphoreType.DMA,
        pltpu.SemaphoreType.DMA,
        pltpu.SemaphoreType.DMA,
        pltpu.SemaphoreType.DMA,
        pltpu.SemaphoreType.DMA,
        pltpu.SemaphoreType.DMA,
    ],
)(_sc_body)


def kernel(table, idx, targets):
    table_p = jnp.pad(table, ((0, 0), (0, VPAD - VOCAB)))
    idx_pad = jnp.pad(idx.reshape(NW, RPT), ((0, 0), (0, RPAD - RPT)))
    tgt_pad = jnp.pad(targets.reshape(NW, RPT), ((0, 0), (0, RPAD - RPT)))
    lse = pl.pallas_call(
        _lse_body,
        out_shape=jax.ShapeDtypeStruct((VPAD, 1), jnp.float32),
    )(table).reshape(-1)
    logits0, tails, partials = _sc_gather(table_p, idx_pad.reshape(-1),
                                          tgt_pad.reshape(-1), lse)
    logits = lax.dynamic_update_slice(logits0, tails[:, :TAIL],
                                      (0, 128 * (NTILE - 1)))
    loss = pl.pallas_call(
        _loss_body,
        out_shape=jax.ShapeDtypeStruct((1, 1), jnp.float32),
    )(partials.reshape(NW, 128))[0, 0]
    return logits, loss


# speedup vs baseline: 1.0363x; 1.0143x over previous
import functools

import jax
import jax.numpy as jnp
from jax import lax
from jax.experimental import pallas as pl
from jax.experimental.pallas import tpu as pltpu
from jax.experimental.pallas import tpu_sc as plsc

VOCAB = 1000
VPAD = 1024
NTILE = VPAD // 128
TAIL = VOCAB - 128 * (NTILE - 1)
NTOK = 1024 * 50
NC, NS, L = 2, 16, 16
NW = NC * NS
RPT = NTOK // NW
RPAD = 2048
CH = 16
NCHUNK = RPT // CH
NQ = NCHUNK // 4


def _lse_body(table_ref, out_ref, pad_ref):
    x = table_ref[...]
    m = jnp.max(x, axis=1, keepdims=True)
    s = jnp.sum(jnp.exp(x - m), axis=1, keepdims=True)
    lse = m + jnp.log(s)
    out_ref[...] = jnp.concatenate(
        [lse, jnp.zeros((VPAD - VOCAB, 1), jnp.float32)], axis=0)
    pad_ref[...] = jnp.concatenate(
        [x, jnp.zeros((VOCAB, VPAD - VOCAB), jnp.float32)], axis=1)


def _loss_body(part_ref, out_ref):
    out_ref[...] = (jnp.sum(part_ref[...]) / NTOK).reshape(1, 1)


def _sc_body(table_hbm, idx_hbm, tgt_hbm, lse_hbm, out_hbm, tails_hbm,
             part_hbm, idx_v, tgt_v, lse_v, buf0, buf1, buf2, buf3, acc_v,
             gsem0, gsem1, gsem2, gsem3, ssem0, ssem1, ssem2, ssem3):
    wid = lax.axis_index("s") * NC + lax.axis_index("c")
    rbase = wid * RPT
    pltpu.sync_copy(idx_hbm.at[pl.ds(wid * RPAD, RPAD)], idx_v)
    pltpu.sync_copy(tgt_hbm.at[pl.ds(wid * RPAD, RPAD)], tgt_v)
    pltpu.sync_copy(lse_hbm, lse_v)

    lane = lax.iota(jnp.int32, L)

    def gather(c, buf, sem):
        return pltpu.make_async_copy(
            table_hbm.at[idx_v.at[pl.ds(c * CH, CH)]], buf, sem)

    def scatters(c, buf, sem):
        r0 = rbase + c * CH
        cps = []
        for t in range(NTILE - 1):
            cps.append(pltpu.make_async_copy(
                buf.at[pl.ds(0, CH), pl.ds(t * 128, 128)],
                out_hbm.at[pl.ds(r0, CH), pl.ds(t * 128, 128)], sem))
        cps.append(pltpu.make_async_copy(
            buf.at[pl.ds(0, CH), pl.ds((NTILE - 1) * 128, 128)],
            tails_hbm.at[pl.ds(r0, CH)], sem))
        return cps

    def start_scatters(c, buf, sem):
        for cp in scatters(c, buf, sem):
            cp.start()

    def wait_scatters(c, buf, sem):
        for cp in scatters(c, buf, sem):
            cp.wait()

    def loss(c, buf, acc):
        for j in range(CH // L):
            lo = c * CH + j * L
            v = idx_v[pl.ds(lo, L)]
            tg = tgt_v[pl.ds(lo, L)]
            trow = lane + j * L
            pick = plsc.load_gather(buf, [trow, tg])
            lseg = plsc.load_gather(lse_v, [v])
            acc = acc + (lseg - pick)
        return acc

    bufs = (buf0, buf1, buf2, buf3)
    gsems = (gsem0, gsem1, gsem2, gsem3)
    ssems = (ssem0, ssem1, ssem2, ssem3)

    gather(0, buf0, gsem0).start()
    gather(1, buf1, gsem1).start()

    def step(k, acc):
        for i in range(4):
            c = 4 * k + i
            n = (i + 2) % 4
            gather(c, bufs[i], gsems[i]).wait()
            if i < 2:
                @pl.when(k > 0)
                def _(c=c, n=n):
                    wait_scatters(c - 2, bufs[n], ssems[n])
                gather(c + 2, bufs[n], gsems[n]).start()
            else:
                wait_scatters(c - 2, bufs[n], ssems[n])

                @pl.when(k < NQ - 1)
                def _(c=c, n=n):
                    gather(c + 2, bufs[n], gsems[n]).start()
            acc = loss(c, bufs[i], acc)
            start_scatters(c, bufs[i], ssems[i])
        return acc

    acc = lax.fori_loop(0, NQ, step, jnp.zeros((L,), jnp.float32))
    wait_scatters(NCHUNK - 2, buf2, ssem2)
    wait_scatters(NCHUNK - 1, buf3, ssem3)
    zero = jnp.zeros((L,), jnp.float32)
    for j in range(8):
        acc_v[pl.ds(j * L, L)] = acc if j == 0 else zero
    pltpu.sync_copy(acc_v, part_hbm.at[pl.ds(wid * 128, 128)])


_sc_gather = functools.partial(
    pl.kernel,
    out_type=[jax.ShapeDtypeStruct((NTOK, VOCAB), jnp.float32),
              jax.ShapeDtypeStruct((NTOK, 128), jnp.float32),
              jax.ShapeDtypeStruct((NW * 128,), jnp.float32)],
    mesh=plsc.VectorSubcoreMesh(core_axis_name="c", subcore_axis_name="s"),
    compiler_params=pltpu.CompilerParams(needs_layout_passes=False,
                                         use_tc_tiling_on_sc=True),
    scratch_types=[
        pltpu.VMEM((RPAD,), jnp.int32),
        pltpu.VMEM((RPAD,), jnp.int32),
        pltpu.VMEM((VPAD,), jnp.float32),
        pltpu.VMEM((CH, VPAD), jnp.float32),
        pltpu.VMEM((CH, VPAD), jnp.float32),
        pltpu.VMEM((CH, VPAD), jnp.float32),
        pltpu.VMEM((CH, VPAD), jnp.float32),
        pltpu.VMEM((128,), jnp.float32),
        pltpu.SemaphoreType.DMA,
        pltpu.SemaphoreType.DMA,
        pltpu.SemaphoreType.DMA,
        pltpu.SemaphoreType.DMA,
        pltpu.SemaphoreType.DMA,
        pltpu.SemaphoreType.DMA,
        pltpu.SemaphoreType.DMA,
        pltpu.SemaphoreType.DMA,
    ],
)(_sc_body)


def kernel(table, idx, targets):
    idx_pad = jnp.pad(idx.reshape(NW, RPT), ((0, 0), (0, RPAD - RPT)))
    tgt_pad = jnp.pad(targets.reshape(NW, RPT), ((0, 0), (0, RPAD - RPT)))
    lse2, table_p = pl.pallas_call(
        _lse_body,
        out_shape=[jax.ShapeDtypeStruct((VPAD, 1), jnp.float32),
                   jax.ShapeDtypeStruct((VOCAB, VPAD), jnp.float32)],
    )(table)
    lse = lse2.reshape(-1)
    logits0, tails, partials = _sc_gather(table_p, idx_pad.reshape(-1),
                                          tgt_pad.reshape(-1), lse)
    logits = lax.dynamic_update_slice(logits0, tails[:, :TAIL],
                                      (0, 128 * (NTILE - 1)))
    loss = pl.pallas_call(
        _loss_body,
        out_shape=jax.ShapeDtypeStruct((1, 1), jnp.float32),
    )(partials.reshape(NW, 128))[0, 0]
    return logits, loss
